# Initial kernel scaffold; baseline (speedup 1.0000x reference)
import jax
import jax.numpy as jnp
from jax.experimental import pallas as pl


def kernel(x_layout, x_role, edge_index, role_emb, layout_emb, W_lin, b_lin, Wl0, bl0, Wr0, Wl1, bl1, Wr1, Wl2, bl2, Wr2, Wd1, bd1, Wd2, bd2, Wd3, bd3):
    raise NotImplementedError("write your pallas kernel here")


# speedup vs baseline: 2.8458x; 2.8458x over previous
import jax
import jax.numpy as jnp
from jax import lax
from jax.experimental import pallas as pl
from jax.experimental.pallas import tpu as pltpu
from jax.experimental.pallas import tpu_sc as plsc

F32 = jnp.float32
I32 = jnp.int32

CHUNK = 2048
BR = 2048

_SC_PARAMS = pltpu.CompilerParams(needs_layout_passes=False,
                                  use_tc_tiling_on_sc=False)


def _tables_body(layp, rolep, wlin, blin, out):
    zpad = jnp.zeros((1056, 80), F32)
    for k in range(4):
        t = jnp.dot(layp[...], wlin[pl.ds(16 * k, 16)],
                    preferred_element_type=F32)
        out[pl.ds(1056 * k, 1056)] = jnp.concatenate([t, zpad], axis=1)
    t = jnp.dot(rolep[...], wlin[pl.ds(64, 16)],
                preferred_element_type=F32) + blin[...]
    out[pl.ds(4224, 128)] = jnp.concatenate([t, zpad[:128]], axis=1)


def _build_tables(layp, rolep, wlin, blin):
    return pl.pallas_call(
        _tables_body,
        out_shape=jax.ShapeDtypeStruct((4352, 128), F32),
    )(layp, rolep, wlin, blin)


def _x0_call(tallv, xlf, xrp, NP):
    NODES_PER_TILE = NP // 32
    X0CH = NODES_PER_TILE // 128
    mesh = plsc.VectorSubcoreMesh(core_axis_name="c", subcore_axis_name="s")

    def body(tallv, xlf, xrp, x0, xlb, xrb, idxb, rows, ob, sem):
        c = lax.axis_index("c")
        s = lax.axis_index("s")
        wid = c * 16 + s
        i16 = lax.iota(I32, 16)
        nbase = wid * NODES_PER_TILE

        def x0_chunk(cc, _):
            n0 = nbase + cc * 128
            pltpu.sync_copy(xlf.at[pl.ds(n0 * 4, 512)], xlb)
            pltpu.sync_copy(xrp.at[pl.ds(n0, 128)], xrb)

            def idx_j(j, _):
                jsl = pl.ds(j * 16, 16)
                lanes = i16 + j * 16
                for k in range(4):
                    pv = lanes * 4 + k
                    v = plsc.load_gather(xlb, [pv])
                    base8 = (v + 1056 * k) * 8
                    for m in range(3):
                        idxb[3 * k + m, jsl] = base8 + m
                rv = xrb[jsl]
                base8 = (rv + 4224) * 8
                for m in range(3):
                    idxb[12 + m, jsl] = base8 + m
                return 0
            lax.fori_loop(0, 8, idx_j, 0)

            def fire_g(g, _):
                pltpu.async_copy(tallv.at[idxb.at[g]], rows.at[g], sem)
                return 0
            lax.fori_loop(0, 15, fire_g, 0)

            def drain_g(g, _):
                pltpu.make_async_copy(tallv.at[idxb.at[g]], rows.at[g],
                                      sem).wait()
                return 0
            lax.fori_loop(0, 15, drain_g, 0)

            def sum_r(r, _):
                for m in range(3):
                    sl = pl.ds(m * 16, 16)
                    ob[r, sl] = ((rows[m, r, :] + rows[3 + m, r, :])
                                 + (rows[6 + m, r, :] + rows[9 + m, r, :])
                                 + rows[12 + m, r, :])
                return 0
            lax.fori_loop(0, 128, sum_r, 0)
            pltpu.sync_copy(ob, x0.at[pl.ds(n0, 128)])
            return 0
        lax.fori_loop(0, X0CH, x0_chunk, 0)

    kfun = pl.kernel(
        body,
        out_type=jax.ShapeDtypeStruct((NP, 128), F32),
        mesh=mesh,
        compiler_params=_SC_PARAMS,
        scratch_types=[
            pltpu.VMEM((512,), I32),
            pltpu.VMEM((128,), I32),
            pltpu.VMEM((15, 128), I32),
            pltpu.VMEM((15, 128, 16), F32),
            pltpu.VMEM((128, 128), F32),
            pltpu.SemaphoreType.DMA,
        ],
    )
    return kfun(tallv, xlf, xrp)


def _deg_call(dstp, NP):
    HALF = NP // 2
    STRIPE = HALF // 16
    ROWS_PER_TILE = dstp.shape[0] // 16
    NCH = ROWS_PER_TILE // 16
    DLR = NP // 16
    MB = DLR // 128
    mesh = plsc.VectorSubcoreMesh(core_axis_name="c", subcore_axis_name="s")

    def body(dstp, degw, degl, dstb, flatb, wbuf, idxsc, acc, semm):
        c = lax.axis_index("c")
        s = lax.axis_index("s")
        i16 = lax.iota(I32, 16)
        zero16 = jnp.zeros((16,), F32)
        one16 = jnp.full((16,), 1.0, F32)

        def zdl(i, _):
            degl[i, :] = zero16
            return 0
        lax.fori_loop(0, DLR, zdl, 0)

        def zwb(i, _):
            wbuf[i, :] = zero16
            return 0
        lax.fori_loop(0, 400, zwb, 0)
        pltpu.sync_copy(wbuf, acc.at[pl.ds(s * (DLR // 16), DLR // 16)])

        def idx_b(b, _):
            def idx_j(j, _):
                idxsc[b, pl.ds(j * 16, 16)] = b * 128 + j * 16 + i16
                return 0
            lax.fori_loop(0, 8, idx_j, 0)
            return 0
        lax.fori_loop(0, MB, idx_b, 0)

        plsc.subcore_barrier()

        def deg_chunk(cc, _):
            pltpu.sync_copy(dstp.at[pl.ds(s * ROWS_PER_TILE + cc * 16, 16)],
                            dstb)

            def loc_i(i, _):
                g = i >> 3
                j = (i & 7) * 16
                d = dstb[g, pl.ds(j, 16)]
                r = jnp.right_shift(d, 4)
                cl = jnp.bitwise_and(d, 15)
                plsc.addupdate_scatter(degl, [r, cl], one16)
                return 0
            lax.fori_loop(0, 128, loc_i, 0)
            return 0
        lax.fori_loop(0, NCH, deg_chunk, 0)

        def fire_m(b, _):
            pltpu.async_copy(degl.at[pl.ds(b * 128, 128)],
                             acc.at[idxsc.at[b]], semm, add=True)
            return 0
        lax.fori_loop(0, MB, fire_m, 0)

        def drain_m(b, _):
            pltpu.make_async_copy(degl.at[pl.ds(b * 128, 128)],
                                  acc.at[idxsc.at[b]], semm).wait()
            return 0
        lax.fori_loop(0, MB, drain_m, 0)
        plsc.subcore_barrier()

        pltpu.sync_copy(acc.at[pl.ds(c * (HALF // 16) + s * (STRIPE // 16),
                                     STRIPE // 16)], flatb)
        zcol = jnp.zeros((16,), I32)
        for w in range(8):
            def exp_q(q, _):
                v = flatb[w * 25 + q, :]
                plsc.store_scatter(wbuf, [q * 16 + i16, zcol], v)
                return 0
            lax.fori_loop(0, 25, exp_q, 0)
            pltpu.sync_copy(
                wbuf, degw.at[pl.ds(c * HALF + s * STRIPE + w * 400, 400)])

    kfun = pl.kernel(
        body,
        out_type=jax.ShapeDtypeStruct((NP, 16), F32),
        mesh=mesh,
        compiler_params=_SC_PARAMS,
        scratch_types=[
            pltpu.VMEM((NP // 16, 16), F32),
            pltpu.VMEM((16, 128), I32),
            pltpu.VMEM((NP // 512, 16), F32),
            pltpu.VMEM((400, 16), F32),
            pltpu.VMEM((NP // 2048, 128), I32),
            pltpu.VMEM_SHARED((NP // 16, 16), F32),
            pltpu.SemaphoreType.DMA,
        ],
    )
    return kfun(dstp)


def _sc_scatter(ytab, srcp, dstp, degw, NP):
    HALF = NP // 2
    STRIPE = HALF // 16
    ROWS_PER_TILE = dstp.shape[0] // 16
    NCH = ROWS_PER_TILE // 16
    WBS = STRIPE // 128
    mesh = plsc.VectorSubcoreMesh(core_axis_name="c", subcore_axis_name="s")

    def body(ytab, srcp, dstp, degw, sview, srcb, dstb, locb, spb, rowsb, zb,
             idxw, acc, semg, sems, semw):
        c = lax.axis_index("c")
        s = lax.axis_index("s")
        half_base = c * HALF
        nodebase = half_base + s * STRIPE
        i16 = lax.iota(I32, 16)
        zero16 = jnp.zeros((16,), F32)

        def zb_init(i, _):
            zb[i, :] = zero16
            return 0
        lax.fori_loop(0, 400, zb_init, 0)

        def scatter_back(p, h):
            def idx_b(b, _):
                def idx_j(j, _):
                    nv = nodebase + h * 640 + b * 128 + j * 16 + i16
                    idxw[b, pl.ds(j * 16, 16)] = nv * 8 + p
                    return 0
                lax.fori_loop(0, 8, idx_j, 0)
                return 0
            lax.fori_loop(0, 5, idx_b, 0)

            def fire_b(b, _):
                pltpu.async_copy(rowsb.at[pl.ds(b * 128, 128)],
                                 sview.at[idxw.at[b]], semw)
                return 0
            lax.fori_loop(0, 5, fire_b, 0)

            def drain_b(b, _):
                pltpu.make_async_copy(rowsb.at[pl.ds(b * 128, 128)],
                                     sview.at[idxw.at[b]], semw).wait()
                return 0
            lax.fori_loop(0, 5, drain_b, 0)

        def deg_h(h, _):
            pltpu.sync_copy(degw.at[pl.ds(nodebase + h * 640, 640)],
                            rowsb.at[pl.ds(0, 640)])
            scatter_back(jnp.int32(3), h)
            return 0
        lax.fori_loop(0, 5, deg_h, 0)

        def pass_p(p, _):
            def zero_k(k, _):
                pltpu.sync_copy(zb, acc.at[pl.ds(s * STRIPE + k * 400, 400)])
                return 0
            lax.fori_loop(0, 8, zero_k, 0)
            plsc.subcore_barrier()

            def chunk(cc, _):
                r0 = s * ROWS_PER_TILE + cc * 16
                pltpu.sync_copy(srcp.at[pl.ds(r0, 16)], srcb)
                pltpu.sync_copy(dstp.at[pl.ds(r0, 16)], dstb)

                def loc_i(i, _):
                    g = i >> 3
                    j = (i & 7) * 16
                    d = dstb[g, pl.ds(j, 16)]
                    loc = d - half_base
                    m = (loc >= 0) & (loc < HALF)
                    locb[g, pl.ds(j, 16)] = jnp.where(m, loc, HALF)
                    spb[g, pl.ds(j, 16)] = srcb[g, pl.ds(j, 16)] * 8 + p
                    return 0
                lax.fori_loop(0, 128, loc_i, 0)

                def fire_g(g, _):
                    pltpu.async_copy(ytab.at[spb.at[g]],
                                     rowsb.at[pl.ds(g * 128, 128)], semg)
                    return 0
                lax.fori_loop(0, 16, fire_g, 0)

                def drain_g(g, _):
                    pltpu.make_async_copy(ytab.at[spb.at[g]],
                                          rowsb.at[pl.ds(g * 128, 128)],
                                          semg).wait()
                    return 0
                lax.fori_loop(0, 16, drain_g, 0)

                def fire_s(g, _):
                    pltpu.async_copy(rowsb.at[pl.ds(g * 128, 128)],
                                     acc.at[locb.at[g]], sems, add=True)
                    return 0
                lax.fori_loop(0, 16, fire_s, 0)

                def drain_s(g, _):
                    pltpu.make_async_copy(rowsb.at[pl.ds(g * 128, 128)],
                                          acc.at[locb.at[g]], sems).wait()
                    return 0
                lax.fori_loop(0, 16, drain_s, 0)
                return 0
            lax.fori_loop(0, NCH, chunk, 0)

            plsc.subcore_barrier()

            def wb_h(h, _):
                pltpu.sync_copy(acc.at[pl.ds(s * STRIPE + h * 640, 640)],
                                rowsb.at[pl.ds(0, 640)])
                scatter_back(p, h)
                return 0
            lax.fori_loop(0, 5, wb_h, 0)
            return 0
        lax.fori_loop(0, 3, pass_p, 0)

    kfun = pl.kernel(
        body,
        out_type=jax.ShapeDtypeStruct((8 * NP, 16), F32),
        mesh=mesh,
        compiler_params=_SC_PARAMS,
        scratch_types=[
            pltpu.VMEM((16, 128), I32),
            pltpu.VMEM((16, 128), I32),
            pltpu.VMEM((16, 128), I32),
            pltpu.VMEM((16, 128), I32),
            pltpu.VMEM((CHUNK, 16), F32),
            pltpu.VMEM((400, 16), F32),
            pltpu.VMEM((5, 128), I32),
            pltpu.VMEM_SHARED((HALF + 8, 16), F32),
            pltpu.SemaphoreType.DMA,
            pltpu.SemaphoreType.DMA,
            pltpu.SemaphoreType.DMA,
        ],
    )
    return kfun(ytab, srcp, dstp, degw)


def _layer_tc(sw, z, wl, wr, bl, NP, x0w=None):
    first = x0w is not None

    def body(*refs):
        if first:
            x0r, wlr, wrr, blr, yout, zout = refs
            x = x0r[...][:, :48]
        else:
            swr, zr, wlr, wrr, blr, yout, zout = refs
            blk = swr[...]
            sm = blk[:, :48]
            d = jnp.maximum(blk[:, 48:49], 1.0)
            x = jnp.maximum(sm / d + zr[...], 0.0)
        y = jnp.dot(x, wlr[...], preferred_element_type=F32)
        yout[...] = jnp.concatenate([y, jnp.zeros((BR, 80), F32)], axis=1)
        zout[...] = jnp.dot(x, wrr[...], preferred_element_type=F32) + blr[...]

    grid = (NP // BR,)
    wspec = pl.BlockSpec((48, 48), lambda i: (0, 0))
    bspec = pl.BlockSpec((1, 48), lambda i: (0, 0))
    io_spec = pl.BlockSpec((BR, 128), lambda i: (i, 0))
    if first:
        in_specs = [io_spec, wspec, wspec, bspec]
        args = (x0w, wl, wr, bl)
    else:
        in_specs = [io_spec, pl.BlockSpec((BR, 48), lambda i: (i, 0)),
                    wspec, wspec, bspec]
        args = (sw, z, wl, wr, bl)
    return pl.pallas_call(
        body,
        grid=grid,
        in_specs=in_specs,
        out_specs=[io_spec, pl.BlockSpec((BR, 48), lambda i: (i, 0))],
        out_shape=[jax.ShapeDtypeStruct((NP, 128), F32),
                   jax.ShapeDtypeStruct((NP, 48), F32)],
    )(*args)


def _final_tc(sw, z, wd1, bd1, wd2, bd2, wd3p, bd3p, NP):
    def body(swr, zr, w1, b1, w2, b2, w3, b3, out):
        blk = swr[...]
        sm = blk[:, :48]
        d = jnp.maximum(blk[:, 48:49], 1.0)
        x = jnp.maximum(sm / d + zr[...], 0.0)
        h = jnp.maximum(jnp.dot(x, w1[...], preferred_element_type=F32)
                        + b1[...], 0.0)
        h = jnp.maximum(jnp.dot(h, w2[...], preferred_element_type=F32)
                        + b2[...], 0.0)
        out[...] = jnp.dot(h, w3[...], preferred_element_type=F32) + b3[...]

    grid = (NP // BR,)
    return pl.pallas_call(
        body,
        grid=grid,
        in_specs=[
            pl.BlockSpec((BR, 128), lambda i: (i, 0)),
            pl.BlockSpec((BR, 48), lambda i: (i, 0)),
            pl.BlockSpec((48, 128), lambda i: (0, 0)),
            pl.BlockSpec((1, 128), lambda i: (0, 0)),
            pl.BlockSpec((128, 128), lambda i: (0, 0)),
            pl.BlockSpec((1, 128), lambda i: (0, 0)),
            pl.BlockSpec((128, 8), lambda i: (0, 0)),
            pl.BlockSpec((1, 8), lambda i: (0, 0)),
        ],
        out_specs=pl.BlockSpec((BR, 8), lambda i: (i, 0)),
        out_shape=jax.ShapeDtypeStruct((NP, 8), F32),
    )(sw, z, wd1, bd1, wd2, bd2, wd3p, bd3p)


def kernel(x_layout, x_role, edge_index, role_emb, layout_emb, W_lin, b_lin,
           Wl0, bl0, Wr0, Wl1, bl1, Wr1, Wl2, bl2, Wr2,
           Wd1, bd1, Wd2, bd2, Wd3, bd3):
    n = x_role.shape[0]
    NP = -(-n // 6400) * 6400
    E = edge_index.shape[1]
    EP = -(-E // (16 * CHUNK)) * (16 * CHUNK)

    src = edge_index[0].astype(I32)
    dst = edge_index[1].astype(I32)
    srcp = jnp.concatenate(
        [src, jnp.zeros((EP - E,), I32)]).reshape(EP // 128, 128)
    dstp = jnp.concatenate(
        [dst, jnp.full((EP - E,), NP - 1, I32)]).reshape(EP // 128, 128)
    xlf = jnp.pad(x_layout.astype(I32), ((0, NP - n), (0, 0))).reshape(NP * 4)
    xrp = jnp.pad(x_role.astype(I32), (0, NP - n))
    layp = jnp.pad(layout_emb, ((0, 1056 - 1025), (0, 0)))
    rolep = jnp.pad(role_emb, ((0, 128 - 120), (0, 0)))

    tall = _build_tables(layp, rolep, W_lin, b_lin.reshape(1, 48))
    x0w = _x0_call(tall.reshape(8 * 4352, 16), xlf, xrp, NP)
    degw = _deg_call(dstp, NP)

    y3k, z = _layer_tc(None, None, Wl0, Wr0, bl0.reshape(1, 48), NP, x0w=x0w)
    sw = _sc_scatter(y3k.reshape(8 * NP, 16), srcp, dstp, degw,
                     NP).reshape(NP, 128)

    wls = jnp.stack([Wl1, Wl2])
    wrs = jnp.stack([Wr1, Wr2])
    bls = jnp.stack([bl1.reshape(1, 48), bl2.reshape(1, 48)])
    two = lax.optimization_barrier(jnp.int32(2))

    def cond(st):
        return st[0] < two

    def step(st):
        i, swc, zc = st
        wl = lax.dynamic_index_in_dim(wls, i, keepdims=False)
        wr = lax.dynamic_index_in_dim(wrs, i, keepdims=False)
        bl = lax.dynamic_index_in_dim(bls, i, keepdims=False)
        y3k2, z2 = _layer_tc(swc, zc, wl, wr, bl, NP)
        sw2 = _sc_scatter(y3k2.reshape(8 * NP, 16), srcp, dstp, degw,
                          NP).reshape(NP, 128)
        return (i + jnp.int32(1), sw2, z2)

    _, sw, z = lax.while_loop(cond, step, (jnp.int32(0), sw, z))

    wd3p = jnp.pad(Wd3, ((0, 0), (0, 7)))
    bd3p = jnp.pad(bd3, (0, 7)).reshape(1, 8)
    out = _final_tc(sw, z, Wd1,
                    bd1.reshape(1, 128), Wd2, bd2.reshape(1, 128),
                    wd3p, bd3p, NP)
    return out[:n, 0:1]
